# baseline (speedup 1.0000x reference)
import jax
import jax.numpy as jnp
import numpy as np
from jax import lax
from jax.experimental import pallas as pl
from jax.experimental.pallas import tpu as pltpu


def _prelu(y, a):
    return jnp.where(y > 0, y, y * a)


def _max3(a, b, c):
    return jnp.maximum(jnp.maximum(a, b), c)


def _shl(v, lanes):
    n = v.shape[-1]
    return jnp.pad(lax.slice_in_dim(v, lanes, n, axis=2),
                   ((0, 0), (0, 0), (0, lanes)))


def _rnet_kernel(x_ref, w1_ref, b1_ref, a1_ref, w2_ref, b2_ref, a2_ref,
                 w3_ref, b3_ref, a3_ref, w4_ref, b4_ref, a4_ref,
                 w5_ref, b5_ref, o_ref):
    B = x_ref.shape[3]
    xq = [x_ref[0, q] for q in range(4)]

    h4 = []
    for q in range(4):
        nrows = 6 if q < 2 else 5
        acc = None
        for dy in range(3):
            r = q + dy
            lhs = xq[r % 4][r // 4:r // 4 + nrows].reshape(nrows * B, 72)
            t = jnp.dot(lhs, w1_ref[dy], preferred_element_type=jnp.float32)
            acc = t if acc is None else acc + t
        h4.append(_prelu(acc + b1_ref[...], a1_ref[0]).reshape(nrows, B, 704))

    rp_e = _max3(h4[0][:5], h4[1][:5], h4[2][:5])
    rp_o = _max3(h4[2][:5], h4[3][:5], h4[0][1:6])
    m_e = _max3(rp_e, _shl(rp_e, 32), _shl(rp_e, 64))
    m_o = _max3(rp_o, _shl(rp_o, 32), _shl(rp_o, 64))

    def conv2(slices):
        acc = None
        for dy, lhs in enumerate(slices):
            t = jnp.dot(lhs.reshape(4 * B, 704), w2_ref[dy],
                        preferred_element_type=jnp.float32)
            acc = t if acc is None else acc + t
        return _prelu(acc + b2_ref[...], a2_ref[0]).reshape(4, B, 512)

    h2e = conv2([m_e[0:4], m_o[0:4], m_e[1:5]])
    h2o = conv2([m_o[0:4], m_e[1:5], m_o[1:5]])

    r2 = _max3(h2e[0:3], h2o[0:3], h2e[1:4])
    m2 = _max3(r2, _shl(r2, 64), _shl(r2, 128))

    acc3 = None
    for hrow in range(3):
        t = jnp.dot(m2[hrow], w3_ref[hrow], preferred_element_type=jnp.float32)
        acc3 = t if acc3 is None else acc3 + t
    h3 = _prelu(acc3 + b3_ref[...], a3_ref[0])
    h4_ = _prelu(jnp.dot(h3, w4_ref[...], preferred_element_type=jnp.float32)
                 + b4_ref[...], a4_ref[0])
    o_ref[...] = (jnp.dot(h4_, w5_ref[...], preferred_element_type=jnp.float32)
                  + b5_ref[...])


def _banded(sel, wv):
    t = jnp.einsum('wjd,ydco->ywcjo', sel, wv)
    d, wi, ci, jo, co = t.shape
    return t.reshape(d, wi * ci, jo * co)


def _sel(wi_n, jo_n, stride):
    s = np.zeros((wi_n, jo_n, 3), np.float32)
    for j in range(jo_n):
        for dx in range(3):
            s[stride * (j + dx), j, dx] = 1.0
    return s


_SEL1 = _sel(24, 22, 1)
_SEL2 = _sel(22, 8, 2)
_P3 = np.zeros((8, 3), np.float32)
for _w in range(3):
    _P3[2 * _w, _w] = 1.0


@jax.jit
def kernel(x_nchw, w1, b1, w2, b2, w3f, b3f, w4, b4, w5, b5, a1, a2, a3, a4):
    N = x_nchw.shape[0]
    B = 256
    Np = ((N + B - 1) // B) * B
    x = x_nchw.astype(jnp.float32)
    if Np != N:
        x = jnp.pad(x, ((0, Np - N), (0, 0), (0, 0), (0, 0)))
    x = jnp.transpose(x.reshape(Np // B, B, 3, 6, 4, 24),
                      (0, 4, 3, 1, 2, 5)).reshape(Np // B, 4, 6, B, 72)

    w1b = jnp.einsum('wjd,ydco->ycwjo', jnp.asarray(_SEL1),
                     w1.reshape(3, 3, 8, 32)[:, :, :3, :]).reshape(3, 72, 704)
    w2b = _banded(jnp.asarray(_SEL2), w2.reshape(3, 3, 32, 64))
    w3r = jnp.einsum('vw,hwco->hvco', jnp.asarray(_P3),
                     w3f.reshape(3, 3, 64, 256)).reshape(3, 512, 256)
    b1t = jnp.tile(b1, 22).reshape(1, 704)
    b2t = jnp.tile(b2, 8).reshape(1, 512)

    smem = pl.BlockSpec(memory_space=pltpu.MemorySpace.SMEM)
    full = lambda shape: pl.BlockSpec(shape, lambda i: (0,) * len(shape))
    out = pl.pallas_call(
        _rnet_kernel,
        out_shape=jax.ShapeDtypeStruct((Np, 15), jnp.float32),
        grid=(Np // B,),
        in_specs=[
            pl.BlockSpec((1, 4, 6, B, 72), lambda i: (i, 0, 0, 0, 0)),
            full((3, 72, 704)), full((1, 704)), smem,
            full((3, 704, 512)), full((1, 512)), smem,
            full((3, 512, 256)), full((1, 256)), smem,
            full((256, 128)), full((1, 128)), smem,
            full((128, 15)), full((1, 15)),
        ],
        out_specs=pl.BlockSpec((B, 15), lambda i: (i, 0)),
        compiler_params=pltpu.CompilerParams(
            dimension_semantics=("parallel",)),
    )(x, w1b, b1t, a1, w2b, b2t, a2, w3r, b3f.reshape(1, 256), a3,
      w4, b4.reshape(1, 128), a4, w5, b5.reshape(1, 15))
    return out[:N] if Np != N else out


# speedup vs baseline: 1.1492x; 1.1492x over previous
import jax
import jax.numpy as jnp
import numpy as np
from jax import lax
from jax.experimental import pallas as pl
from jax.experimental.pallas import tpu as pltpu


def _prelu(y, a):
    return jnp.where(y > 0, y, y * a)


def _max3(a, b, c):
    return jnp.maximum(jnp.maximum(a, b), c)


def _shl(v, lanes):
    n = v.shape[-1]
    return jnp.pad(lax.slice_in_dim(v, lanes, n, axis=2),
                   ((0, 0), (0, 0), (0, lanes)))


def _rnet_kernel(x_ref, w1_ref, b1_ref, a1_ref, w2_ref, b2_ref, a2_ref,
                 w3_ref, b3_ref, a3_ref, w4_ref, b4_ref, a4_ref,
                 w5_ref, b5_ref, o_ref):
    B = x_ref.shape[3]
    xq = [x_ref[0, q] for q in range(4)]

    h4 = []
    for q in range(4):
        nrows = 6 if q < 2 else 5
        acc = None
        for dy in range(3):
            r = q + dy
            lhs = xq[r % 4][r // 4:r // 4 + nrows].reshape(nrows * B, 72)
            t = jnp.dot(lhs, w1_ref[dy], preferred_element_type=jnp.float32)
            acc = t if acc is None else acc + t
        h4.append(_prelu(acc + b1_ref[...], a1_ref[0])
                  .astype(jnp.bfloat16).reshape(nrows, B, 704))

    rp_e = _max3(h4[0][:5], h4[1][:5], h4[2][:5])
    rp_o = _max3(h4[2][:5], h4[3][:5], h4[0][1:6])
    m_e = _max3(rp_e, _shl(rp_e, 32), _shl(rp_e, 64))
    m_o = _max3(rp_o, _shl(rp_o, 32), _shl(rp_o, 64))

    def conv2(slices):
        acc = None
        for dy, lhs in enumerate(slices):
            t = jnp.dot(lhs.reshape(4 * B, 704), w2_ref[dy],
                        preferred_element_type=jnp.float32)
            acc = t if acc is None else acc + t
        return (_prelu(acc + b2_ref[...], a2_ref[0])
                .astype(jnp.bfloat16).reshape(4, B, 512))

    h2e = conv2([m_e[0:4], m_o[0:4], m_e[1:5]])
    h2o = conv2([m_o[0:4], m_e[1:5], m_o[1:5]])

    r2 = _max3(h2e[0:3], h2o[0:3], h2e[1:4])
    m2 = _max3(r2, _shl(r2, 64), _shl(r2, 128))

    acc3 = None
    for hrow in range(3):
        t = jnp.dot(m2[hrow], w3_ref[hrow], preferred_element_type=jnp.float32)
        acc3 = t if acc3 is None else acc3 + t
    h3 = _prelu(acc3 + b3_ref[...], a3_ref[0]).astype(jnp.bfloat16)
    h4_ = _prelu(jnp.dot(h3, w4_ref[...], preferred_element_type=jnp.float32)
                 + b4_ref[...], a4_ref[0]).astype(jnp.bfloat16)
    o_ref[...] = (jnp.dot(h4_, w5_ref[...], preferred_element_type=jnp.float32)
                  + b5_ref[...])


def _banded(sel, wv):
    t = jnp.einsum('wjd,ydco->ywcjo', sel, wv)
    d, wi, ci, jo, co = t.shape
    return t.reshape(d, wi * ci, jo * co)


def _sel(wi_n, jo_n, stride):
    s = np.zeros((wi_n, jo_n, 3), np.float32)
    for j in range(jo_n):
        for dx in range(3):
            s[stride * (j + dx), j, dx] = 1.0
    return s


_SEL1 = _sel(24, 22, 1)
_SEL2 = _sel(22, 8, 2)
_P3 = np.zeros((8, 3), np.float32)
for _w in range(3):
    _P3[2 * _w, _w] = 1.0


@jax.jit
def kernel(x_nchw, w1, b1, w2, b2, w3f, b3f, w4, b4, w5, b5, a1, a2, a3, a4):
    N = x_nchw.shape[0]
    B = 256
    Np = ((N + B - 1) // B) * B
    x = x_nchw.astype(jnp.bfloat16)
    if Np != N:
        x = jnp.pad(x, ((0, Np - N), (0, 0), (0, 0), (0, 0)))
    x = jnp.transpose(x.reshape(Np // B, B, 3, 6, 4, 24),
                      (0, 4, 3, 1, 2, 5)).reshape(Np // B, 4, 6, B, 72)

    w1b = jnp.einsum('wjd,ydco->ycwjo', jnp.asarray(_SEL1),
                     w1.reshape(3, 3, 8, 32)[:, :, :3, :]).reshape(
                         3, 72, 704).astype(jnp.bfloat16)
    w2b = _banded(jnp.asarray(_SEL2),
                  w2.reshape(3, 3, 32, 64)).astype(jnp.bfloat16)
    w3r = jnp.einsum('vw,hwco->hvco', jnp.asarray(_P3),
                     w3f.reshape(3, 3, 64, 256)).reshape(
                         3, 512, 256).astype(jnp.bfloat16)
    b1t = jnp.tile(b1, 22).reshape(1, 704)
    b2t = jnp.tile(b2, 8).reshape(1, 512)

    smem = pl.BlockSpec(memory_space=pltpu.MemorySpace.SMEM)
    full = lambda shape: pl.BlockSpec(shape, lambda i: (0,) * len(shape))
    out = pl.pallas_call(
        _rnet_kernel,
        out_shape=jax.ShapeDtypeStruct((Np, 15), jnp.float32),
        grid=(Np // B,),
        in_specs=[
            pl.BlockSpec((1, 4, 6, B, 72), lambda i: (i, 0, 0, 0, 0)),
            full((3, 72, 704)), full((1, 704)), smem,
            full((3, 704, 512)), full((1, 512)), smem,
            full((3, 512, 256)), full((1, 256)), smem,
            full((256, 128)), full((1, 128)), smem,
            full((128, 15)), full((1, 15)),
        ],
        out_specs=pl.BlockSpec((B, 15), lambda i: (i, 0)),
        compiler_params=pltpu.CompilerParams(
            dimension_semantics=("parallel",)),
    )(x, w1b, b1t, a1, w2b, b2t, a2, w3r, b3f.reshape(1, 256), a3,
      w4.astype(jnp.bfloat16), b4.reshape(1, 128), a4,
      w5.astype(jnp.bfloat16), b5.reshape(1, 15))
    return out[:N] if Np != N else out


# speedup vs baseline: 1.1540x; 1.0042x over previous
import jax
import jax.numpy as jnp
import numpy as np
from jax import lax
from jax.experimental import pallas as pl
from jax.experimental.pallas import tpu as pltpu


def _prelu(y, a):
    return jnp.where(y > 0, y, y * a)


def _max3(a, b, c):
    return jnp.maximum(jnp.maximum(a, b), c)


def _shl(v, lanes):
    n = v.shape[-1]
    return jnp.pad(lax.slice_in_dim(v, lanes, n, axis=2),
                   ((0, 0), (0, 0), (0, lanes)))


def _rnet_kernel(x_ref, w1_ref, b1_ref, a1_ref, w2_ref, b2_ref, a2_ref,
                 w3_ref, b3_ref, a3_ref, w4_ref, b4_ref, a4_ref,
                 w5_ref, b5_ref, o_ref):
    B = x_ref.shape[3]
    xq = [x_ref[0, q] for q in range(4)]

    h4 = []
    for q in range(4):
        nrows = 6 if q < 2 else 5
        acc = None
        for dy in range(3):
            r = q + dy
            lhs = xq[r % 4][r // 4:r // 4 + nrows].reshape(nrows * B, 72)
            t = jnp.dot(lhs, w1_ref[dy], preferred_element_type=jnp.float32)
            acc = t if acc is None else acc + t
        h4.append(_prelu(acc + b1_ref[...], a1_ref[0])
                  .astype(jnp.bfloat16).reshape(nrows, B, 704))

    rp_e = _max3(h4[0][:5], h4[1][:5], h4[2][:5])
    rp_o = _max3(h4[2][:5], h4[3][:5], h4[0][1:6])
    m_e = _max3(rp_e, _shl(rp_e, 32), _shl(rp_e, 64))
    m_o = _max3(rp_o, _shl(rp_o, 32), _shl(rp_o, 64))

    def conv2(slices):
        acc = None
        for dy, lhs in enumerate(slices):
            t = jnp.dot(lhs.reshape(4 * B, 704), w2_ref[dy],
                        preferred_element_type=jnp.float32)
            acc = t if acc is None else acc + t
        return (_prelu(acc + b2_ref[...], a2_ref[0])
                .astype(jnp.bfloat16).reshape(4, B, 512))

    h2e = conv2([m_e[0:4], m_o[0:4], m_e[1:5]])
    h2o = conv2([m_o[0:4], m_e[1:5], m_o[1:5]])

    r2 = _max3(h2e[0:3], h2o[0:3], h2e[1:4])
    m2 = _max3(r2, _shl(r2, 64), _shl(r2, 128))

    acc3 = None
    for hrow in range(3):
        t = jnp.dot(m2[hrow], w3_ref[hrow], preferred_element_type=jnp.float32)
        acc3 = t if acc3 is None else acc3 + t
    h3 = _prelu(acc3 + b3_ref[...], a3_ref[0]).astype(jnp.bfloat16)
    h4_ = _prelu(jnp.dot(h3, w4_ref[...], preferred_element_type=jnp.float32)
                 + b4_ref[...], a4_ref[0]).astype(jnp.bfloat16)
    o_ref[...] = (jnp.dot(h4_, w5_ref[...], preferred_element_type=jnp.float32)
                  + b5_ref[...])


def _banded(sel, wv):
    t = jnp.einsum('wjd,ydco->ywcjo', sel, wv)
    d, wi, ci, jo, co = t.shape
    return t.reshape(d, wi * ci, jo * co)


def _sel(wi_n, jo_n, stride):
    s = np.zeros((wi_n, jo_n, 3), np.float32)
    for j in range(jo_n):
        for dx in range(3):
            s[stride * (j + dx), j, dx] = 1.0
    return s


_SEL1 = _sel(24, 22, 1)
_SEL2 = _sel(22, 8, 2)
_P3 = np.zeros((8, 3), np.float32)
for _w in range(3):
    _P3[2 * _w, _w] = 1.0


@jax.jit
def kernel(x_nchw, w1, b1, w2, b2, w3f, b3f, w4, b4, w5, b5, a1, a2, a3, a4):
    N = x_nchw.shape[0]
    B = 512
    Np = ((N + B - 1) // B) * B
    x = x_nchw.astype(jnp.bfloat16)
    if Np != N:
        x = jnp.pad(x, ((0, Np - N), (0, 0), (0, 0), (0, 0)))
    x = jnp.transpose(x.reshape(Np // B, B, 3, 6, 4, 24),
                      (0, 4, 3, 1, 2, 5)).reshape(Np // B, 4, 6, B, 72)

    w1b = jnp.einsum('wjd,ydco->ycwjo', jnp.asarray(_SEL1),
                     w1.reshape(3, 3, 8, 32)[:, :, :3, :]).reshape(
                         3, 72, 704).astype(jnp.bfloat16)
    w2b = _banded(jnp.asarray(_SEL2),
                  w2.reshape(3, 3, 32, 64)).astype(jnp.bfloat16)
    w3r = jnp.einsum('vw,hwco->hvco', jnp.asarray(_P3),
                     w3f.reshape(3, 3, 64, 256)).reshape(
                         3, 512, 256).astype(jnp.bfloat16)
    b1t = jnp.tile(b1, 22).reshape(1, 704)
    b2t = jnp.tile(b2, 8).reshape(1, 512)

    smem = pl.BlockSpec(memory_space=pltpu.MemorySpace.SMEM)
    full = lambda shape: pl.BlockSpec(shape, lambda i: (0,) * len(shape))
    out = pl.pallas_call(
        _rnet_kernel,
        out_shape=jax.ShapeDtypeStruct((Np, 15), jnp.float32),
        grid=(Np // B,),
        in_specs=[
            pl.BlockSpec((1, 4, 6, B, 72), lambda i: (i, 0, 0, 0, 0)),
            full((3, 72, 704)), full((1, 704)), smem,
            full((3, 704, 512)), full((1, 512)), smem,
            full((3, 512, 256)), full((1, 256)), smem,
            full((256, 128)), full((1, 128)), smem,
            full((128, 15)), full((1, 15)),
        ],
        out_specs=pl.BlockSpec((B, 15), lambda i: (i, 0)),
        compiler_params=pltpu.CompilerParams(
            dimension_semantics=("parallel",)),
    )(x, w1b, b1t, a1, w2b, b2t, a2, w3r, b3f.reshape(1, 256), a3,
      w4.astype(jnp.bfloat16), b4.reshape(1, 128), a4,
      w5.astype(jnp.bfloat16), b5.reshape(1, 15))
    return out[:N] if Np != N else out


# speedup vs baseline: 1.2109x; 1.0493x over previous
import jax
import jax.numpy as jnp
import numpy as np
from jax import lax
from jax.experimental import pallas as pl
from jax.experimental.pallas import tpu as pltpu


def _prelu(y, a):
    return jnp.where(y > 0, y, y * a)


def _max3(a, b, c):
    return jnp.maximum(jnp.maximum(a, b), c)


def _shl(v, lanes):
    n = v.shape[-1]
    return jnp.pad(lax.slice_in_dim(v, lanes, n, axis=2),
                   ((0, 0), (0, 0), (0, lanes)))


def _rnet_kernel(x_ref, w1_ref, b1_ref, a1_ref, w2lo_ref, w2hi_ref, b2_ref,
                 a2_ref, w3_ref, b3_ref, a3_ref, w4_ref, b4_ref, a4_ref,
                 w5_ref, b5_ref, o_ref):
    B = x_ref.shape[3]
    xq = [x_ref[0, q] for q in range(4)]

    h4 = []
    for q in range(4):
        nrows = 6 if q < 2 else 5
        acc = None
        for dy in range(3):
            r = q + dy
            lhs = xq[r % 4][r // 4:r // 4 + nrows].reshape(nrows * B, 72)
            t = jnp.dot(lhs, w1_ref[dy], preferred_element_type=jnp.float32)
            acc = t if acc is None else acc + t
        h4.append(_prelu(acc + b1_ref[...], a1_ref[0])
                  .astype(jnp.bfloat16).reshape(nrows, B, 704))

    rp_e = _max3(h4[0][:5], h4[1][:5], h4[2][:5])
    rp_o = _max3(h4[2][:5], h4[3][:5], h4[0][1:6])
    m_e = _max3(rp_e, _shl(rp_e, 32), _shl(rp_e, 64))
    m_o = _max3(rp_o, _shl(rp_o, 32), _shl(rp_o, 64))

    def conv2(slices):
        lo = hi = None
        for dy, lhs in enumerate(slices):
            l2 = lhs.reshape(4 * B, 704)
            tl = jnp.dot(l2[:, 0:384], w2lo_ref[dy],
                         preferred_element_type=jnp.float32)
            th = jnp.dot(l2[:, 256:640], w2hi_ref[dy],
                         preferred_element_type=jnp.float32)
            lo = tl if lo is None else lo + tl
            hi = th if hi is None else hi + th
        acc = jnp.concatenate([lo, hi], axis=-1)
        return (_prelu(acc + b2_ref[...], a2_ref[0])
                .astype(jnp.bfloat16).reshape(4, B, 512))

    h2e = conv2([m_e[0:4], m_o[0:4], m_e[1:5]])
    h2o = conv2([m_o[0:4], m_e[1:5], m_o[1:5]])

    r2 = _max3(h2e[0:3], h2o[0:3], h2e[1:4])
    m2 = _max3(r2, _shl(r2, 64), _shl(r2, 128))

    acc3 = None
    for hrow in range(3):
        t = jnp.dot(m2[hrow], w3_ref[hrow], preferred_element_type=jnp.float32)
        acc3 = t if acc3 is None else acc3 + t
    h3 = _prelu(acc3 + b3_ref[...], a3_ref[0]).astype(jnp.bfloat16)
    h4_ = _prelu(jnp.dot(h3, w4_ref[...], preferred_element_type=jnp.float32)
                 + b4_ref[...], a4_ref[0]).astype(jnp.bfloat16)
    o_ref[...] = (jnp.dot(h4_, w5_ref[...], preferred_element_type=jnp.float32)
                  + b5_ref[...])


def _banded(sel, wv):
    t = jnp.einsum('wjd,ydco->ywcjo', sel, wv)
    d, wi, ci, jo, co = t.shape
    return t.reshape(d, wi * ci, jo * co)


def _sel(wi_n, jo_n, stride):
    s = np.zeros((wi_n, jo_n, 3), np.float32)
    for j in range(jo_n):
        for dx in range(3):
            s[stride * (j + dx), j, dx] = 1.0
    return s


_SEL1 = _sel(24, 22, 1)
_SEL2 = _sel(22, 8, 2)
_P3 = np.zeros((8, 3), np.float32)
for _w in range(3):
    _P3[2 * _w, _w] = 1.0


@jax.jit
def kernel(x_nchw, w1, b1, w2, b2, w3f, b3f, w4, b4, w5, b5, a1, a2, a3, a4):
    N = x_nchw.shape[0]
    B = 512
    Np = ((N + B - 1) // B) * B
    x = x_nchw.astype(jnp.bfloat16)
    if Np != N:
        x = jnp.pad(x, ((0, Np - N), (0, 0), (0, 0), (0, 0)))
    x = jnp.transpose(x.reshape(Np // B, B, 3, 6, 4, 24),
                      (0, 4, 3, 1, 2, 5)).reshape(Np // B, 4, 6, B, 72)

    w1b = jnp.einsum('wjd,ydco->ycwjo', jnp.asarray(_SEL1),
                     w1.reshape(3, 3, 8, 32)[:, :, :3, :]).reshape(
                         3, 72, 704).astype(jnp.bfloat16)
    w2b = _banded(jnp.asarray(_SEL2),
                  w2.reshape(3, 3, 32, 64)).astype(jnp.bfloat16)
    w2lo = w2b[:, 0:384, 0:256]
    w2hi = w2b[:, 256:640, 256:512]
    w3r = jnp.einsum('vw,hwco->hvco', jnp.asarray(_P3),
                     w3f.reshape(3, 3, 64, 256)).reshape(
                         3, 512, 256).astype(jnp.bfloat16)
    b1t = jnp.tile(b1, 22).reshape(1, 704)
    b2t = jnp.tile(b2, 8).reshape(1, 512)

    smem = pl.BlockSpec(memory_space=pltpu.MemorySpace.SMEM)
    full = lambda shape: pl.BlockSpec(shape, lambda i: (0,) * len(shape))
    out = pl.pallas_call(
        _rnet_kernel,
        out_shape=jax.ShapeDtypeStruct((Np, 15), jnp.float32),
        grid=(Np // B,),
        in_specs=[
            pl.BlockSpec((1, 4, 6, B, 72), lambda i: (i, 0, 0, 0, 0)),
            full((3, 72, 704)), full((1, 704)), smem,
            full((3, 384, 256)), full((3, 384, 256)), full((1, 512)), smem,
            full((3, 512, 256)), full((1, 256)), smem,
            full((256, 128)), full((1, 128)), smem,
            full((128, 15)), full((1, 15)),
        ],
        out_specs=pl.BlockSpec((B, 15), lambda i: (i, 0)),
        compiler_params=pltpu.CompilerParams(
            dimension_semantics=("parallel",)),
    )(x, w1b, b1t, a1, w2lo, w2hi, b2t, a2, w3r, b3f.reshape(1, 256), a3,
      w4.astype(jnp.bfloat16), b4.reshape(1, 128), a4,
      w5.astype(jnp.bfloat16), b5.reshape(1, 15))
    return out[:N] if Np != N else out
